# baseline (speedup 1.0000x reference)
import functools

import jax
import jax.numpy as jnp
from jax import lax
from jax.experimental import pallas as pl
from jax.experimental.pallas import tpu as pltpu
from jax.experimental.pallas import tpu_sc as plsc

N_NODES = 10000
N_HEDGES = 5000
N_INC = 320000
D = 128
HEADS = 8

NC = 2
NS = 16
NW = NC * NS

E_PAD = 5120
E_ROWS = E_PAD // NS
N_PAD = 10240
AR2 = N_PAD + 8
V_ROWS = N_PAD // NS

K = 80
SCH = 25
PER_TILE = N_INC // NW
NCH1 = PER_TILE // K
NSC1 = NCH1 // SCH
PAD2 = 240
NCH2 = (PER_TILE + PAD2) // K
SCH2 = 16
NSC2 = NCH2 // SCH2
DEPTH2 = 4

_SC_PARAMS = pltpu.CompilerParams(needs_layout_passes=False)


@functools.lru_cache(maxsize=None)
def _get_mesh():
  return plsc.VectorSubcoreMesh(
      core_axis_name="c", subcore_axis_name="s", num_cores=NC, num_subcores=NS)


DEPTH = 5


def _ring_loop(src_hbm, gidx, sidx, acc, bufs, gsems, ssems, nch,
               pre=None, post=None):
  depth = len(bufs)

  def g_issue(j, q):
    pltpu.async_copy(src_hbm.at[gidx.at[j]], bufs[q], gsems[q])

  def g_wait(q):
    pltpu.make_async_copy(src_hbm.at[gidx.at[0]], bufs[q], gsems[q]).wait()

  def s_wait(q):
    pltpu.make_async_copy(bufs[q], acc.at[sidx.at[0]], ssems[q]).wait()

  g_issue(0, 0)
  g_issue(1, 1)

  def body(i, carry):
    for q in range(depth):
      j = depth * i + q
      g_wait(q)
      if pre is not None:
        pre(j)
      pltpu.async_copy(bufs[q], acc.at[sidx.at[j]], ssems[q], add=True)
      q2 = (q + 2) % depth
      pl.when(j >= depth - 2)(lambda q2=q2: s_wait(q2))
      pl.when(j + 2 < nch)(lambda j=j, q2=q2: g_issue(j + 2, q2))
      if post is not None:
        post(j)
    return carry

  lax.fori_loop(0, nch // depth, body, 0)
  for dq in range(depth - 2, 0, -1):
    s_wait((nch - dq) % depth)


def _stage1_body(x_hbm, v4_hbm, e4_hbm, zrow_hbm, zflat_hbm,
                 outp_hbm, outce_hbm, outcv_hbm,
                 acc, vloc, eloc, b0, b1, b2, b3, b4, cnt_e, cnt_v,
                 g0, g1, g2, g3, g4, s0, s1, s2, s3, s4):
  cid = lax.axis_index("c")
  sid = lax.axis_index("s")
  wid = cid * NS + sid
  bufs = (b0, b1, b2, b3, b4)
  gsems = (g0, g1, g2, g3, g4)
  ssems = (s0, s1, s2, s3, s4)
  sl = pl.ds(sid * E_ROWS, E_ROWS)
  pltpu.sync_copy(zrow_hbm.at[pl.ds(0, E_ROWS)], acc.at[sl])
  pltpu.sync_copy(zflat_hbm.at[pl.ds(0, E_PAD)], cnt_e)
  pltpu.sync_copy(zflat_hbm, cnt_v)
  plsc.subcore_barrier()

  ones16 = jnp.ones((16,), jnp.float32)

  def hist(j):
    for t in range(K // 16):
      tsl = pl.ds(t * 16, 16)
      plsc.addupdate_scatter(cnt_e, [eloc[j, tsl]], ones16)
      plsc.addupdate_scatter(cnt_v, [vloc[j, tsl]], ones16)

  def sbody(sc, carry):
    pltpu.sync_copy(v4_hbm.at[wid, sc], vloc)
    pltpu.sync_copy(e4_hbm.at[wid, sc], eloc)
    _ring_loop(x_hbm, vloc, eloc, acc, bufs, gsems, ssems, SCH, post=hist)
    return carry

  lax.fori_loop(0, NSC1, sbody, 0)
  plsc.subcore_barrier()
  pltpu.sync_copy(acc.at[sl], outp_hbm.at[cid, sl])
  pltpu.sync_copy(cnt_e, outce_hbm.at[cid, sid])
  pltpu.sync_copy(cnt_v, outcv_hbm.at[cid, sid])


@functools.lru_cache(maxsize=None)
def _make_stage1():
  return functools.partial(
      pl.kernel,
      mesh=_get_mesh(),
      compiler_params=_SC_PARAMS,
      out_type=(
          jax.ShapeDtypeStruct((NC, E_PAD, D), jnp.float32),
          jax.ShapeDtypeStruct((NC, NS, E_PAD), jnp.float32),
          jax.ShapeDtypeStruct((NC, NS, N_PAD), jnp.float32),
      ),
      scratch_types=(
          [pltpu.VMEM_SHARED((E_PAD, D), jnp.float32),
           pltpu.VMEM((SCH, K), jnp.int32),
           pltpu.VMEM((SCH, K), jnp.int32)]
          + [pltpu.VMEM((K, D), jnp.float32)] * DEPTH
          + [pltpu.VMEM((E_PAD,), jnp.float32),
             pltpu.VMEM((N_PAD,), jnp.float32)]
          + [pltpu.SemaphoreType.DMA] * (2 * DEPTH)
      ),
  )(_stage1_body)


def _stage2_body(xe_hbm, e4_hbm, v4_hbm, zrow_hbm,
                 outp_hbm,
                 acc, gloc, sloc, b0, b1, b2, b3,
                 g0, g1, g2, g3, s0, s1, s2, s3):
  cid = lax.axis_index("c")
  sid = lax.axis_index("s")
  wid = cid * NS + sid
  bufs = (b0, b1, b2, b3)
  gsems = (g0, g1, g2, g3)
  ssems = (s0, s1, s2, s3)
  for q in range(2):
    qsl = pl.ds(sid * V_ROWS + q * (V_ROWS // 2), V_ROWS // 2)
    pltpu.sync_copy(zrow_hbm.at[pl.ds(0, V_ROWS // 2)], acc.at[qsl])
  plsc.subcore_barrier()

  def sbody(sc, carry):
    pltpu.sync_copy(e4_hbm.at[wid, sc], gloc)
    pltpu.sync_copy(v4_hbm.at[wid, sc], sloc)
    _ring_loop(xe_hbm, gloc, sloc, acc, bufs, gsems, ssems, SCH2)
    return carry

  lax.fori_loop(0, NSC2, sbody, 0)
  plsc.subcore_barrier()
  for q in range(2):
    qsl = pl.ds(sid * V_ROWS + q * (V_ROWS // 2), V_ROWS // 2)
    pltpu.sync_copy(acc.at[qsl], outp_hbm.at[cid, qsl])


@functools.lru_cache(maxsize=None)
def _make_stage2():
  return functools.partial(
      pl.kernel,
      mesh=_get_mesh(),
      compiler_params=_SC_PARAMS,
      out_type=jax.ShapeDtypeStruct((NC, N_PAD, D), jnp.float32),
      scratch_types=(
          [pltpu.VMEM_SHARED((AR2, D), jnp.float32),
           pltpu.VMEM((SCH2, K), jnp.int32),
           pltpu.VMEM((SCH2, K), jnp.int32)]
          + [pltpu.VMEM((K, D), jnp.float32)] * DEPTH2
          + [pltpu.SemaphoreType.DMA] * (2 * DEPTH2)
      ),
  )(_stage2_body)


def _combine_body(p_ref, c_ref, o_ref):
  p = p_ref[0] + p_ref[1]
  c = jnp.sum(c_ref[...], axis=(0, 1))[:, None]
  o_ref[...] = p / jnp.maximum(c, 1.0)


def _combine(edge_p, edge_c):
  blk = 512
  grid = E_PAD // blk
  return pl.pallas_call(
      _combine_body,
      grid=(grid,),
      in_specs=[
          pl.BlockSpec((NC, blk, D), lambda i: (0, i, 0)),
          pl.BlockSpec((NC, NS, blk), lambda i: (0, 0, i)),
      ],
      out_specs=pl.BlockSpec((blk, D), lambda i: (i, 0)),
      out_shape=jax.ShapeDtypeStruct((E_PAD, D), jnp.float32),
  )(edge_p, edge_c)


def _linear_body(x_ref, w_ref, b_ref, y_ref):
  y_ref[...] = (jnp.dot(x_ref[...], w_ref[...],
                        preferred_element_type=jnp.float32) + b_ref[...])


def _linear(x, w_t, b2d):
  blk = 1000
  grid = N_NODES // blk
  return pl.pallas_call(
      _linear_body,
      grid=(grid,),
      in_specs=[
          pl.BlockSpec((blk, D), lambda i: (i, 0)),
          pl.BlockSpec((D, D), lambda i: (0, 0)),
          pl.BlockSpec((1, D), lambda i: (0, 0)),
      ],
      out_specs=pl.BlockSpec((blk, D), lambda i: (i, 0)),
      out_shape=jax.ShapeDtypeStruct((N_NODES, D), jnp.float32),
  )(x, w_t, b2d)


def _final_body(p_ref, c_ref, a_ref, x_ref, loss_ref):
  c = jnp.sum(c_ref[...], axis=(0, 1))[:, None]
  x = jnp.maximum((p_ref[0] + p_ref[1]) / jnp.maximum(c, 1.0), 0.0)
  x_ref[...] = x

  blk_rows = x.shape[0]
  att = jnp.tile(a_ref[...], (blk_rows // HEADS, 1))
  a = jnp.sum(x * att, axis=1, keepdims=True) * (1.0 / D)
  a = jnp.where(a >= 0.0, a, 0.2 * a)
  s = jnp.clip(jax.nn.sigmoid(a), 0.01, 0.99)
  kl = s * jnp.log(s / 0.5) + (1.0 - s) * jnp.log((1.0 - s) / 0.5)
  part = jnp.sum(kl).reshape(1, 1)

  @pl.when(pl.program_id(0) == 0)
  def _():
    loss_ref[...] = jnp.zeros((1, 1), jnp.float32)

  loss_ref[...] += part


def _finalize(node_p, node_c, att):
  blk = 1024
  grid = N_PAD // blk
  return pl.pallas_call(
      _final_body,
      grid=(grid,),
      in_specs=[
          pl.BlockSpec((NC, blk, D), lambda i: (0, i, 0)),
          pl.BlockSpec((NC, NS, blk), lambda i: (0, 0, i)),
          pl.BlockSpec((HEADS, D), lambda i: (0, 0)),
      ],
      out_specs=[
          pl.BlockSpec((blk, D), lambda i: (i, 0)),
          pl.BlockSpec((1, 1), lambda i: (0, 0)),
      ],
      out_shape=[
          jax.ShapeDtypeStruct((N_PAD, D), jnp.float32),
          jax.ShapeDtypeStruct((1, 1), jnp.float32),
      ],
  )(node_p, node_c, att)


def kernel(X, v_idx, e_idx, theta_W, theta_b, att):
  v_idx = v_idx.astype(jnp.int32)
  e_idx = e_idx.astype(jnp.int32)
  zrow = jnp.zeros((E_ROWS, D), jnp.float32)
  zflat = jnp.zeros((N_PAD,), jnp.float32)
  v31 = v_idx.reshape(NW, NSC1, SCH, K)
  e31 = e_idx.reshape(NW, NSC1, SCH, K)
  vpad = jnp.broadcast_to(N_PAD + (jnp.arange(PAD2, dtype=jnp.int32) % 8),
                          (NW, PAD2))
  epad = jnp.zeros((NW, PAD2), jnp.int32)
  v32 = jnp.concatenate([v_idx.reshape(NW, PER_TILE), vpad],
                        axis=1).reshape(NW, NSC2, SCH2, K)
  e32 = jnp.concatenate([e_idx.reshape(NW, PER_TILE), epad],
                        axis=1).reshape(NW, NSC2, SCH2, K)

  y = _linear(X, theta_W.T, theta_b[None, :])
  edge_p, edge_c, node_c = _make_stage1()(y, v31, e31, zrow, zflat)
  x_e = _combine(edge_p, edge_c)
  node_p = _make_stage2()(x_e, e32, v32, zrow)
  x_out, loss = _finalize(node_p, node_c, att)
  return x_out[:N_NODES], loss[0, 0]


# speedup vs baseline: 1.4679x; 1.4679x over previous
import functools

import jax
import jax.numpy as jnp
from jax import lax
from jax.experimental import pallas as pl
from jax.experimental.pallas import tpu as pltpu
from jax.experimental.pallas import tpu_sc as plsc

N_NODES = 10000
N_HEDGES = 5000
N_INC = 320000
D = 128
HEADS = 8

NC = 2
NS = 16
NW = NC * NS

E_PAD = 5120
E_ROWS = E_PAD // NS
N_PAD = 10240
HALF = N_PAD // 2
TRASH = 512
AR = HALF + TRASH
A_ROWS = AR // NS

K = 80
SCH = 25
PER_TILE = N_INC // NW
NCH1 = PER_TILE // K
NSC1 = NCH1 // SCH
PER_TILE2 = N_INC // NS
NCH2 = PER_TILE2 // K
NSC2 = NCH2 // SCH

_SC_PARAMS = pltpu.CompilerParams(needs_layout_passes=False)


@functools.lru_cache(maxsize=None)
def _get_mesh():
  return plsc.VectorSubcoreMesh(
      core_axis_name="c", subcore_axis_name="s", num_cores=NC, num_subcores=NS)


DEPTH = 5


def _ring_loop(src_hbm, gidx, sidx, acc, bufs, gsems, ssems, nch,
               pre=None, post=None):
  depth = len(bufs)

  def g_issue(j, q):
    pltpu.async_copy(src_hbm.at[gidx.at[j]], bufs[q], gsems[q])

  def g_wait(q):
    pltpu.make_async_copy(src_hbm.at[gidx.at[0]], bufs[q], gsems[q]).wait()

  def s_wait(q):
    pltpu.make_async_copy(bufs[q], acc.at[sidx.at[0]], ssems[q]).wait()

  g_issue(0, 0)
  g_issue(1, 1)

  def body(i, carry):
    for q in range(depth):
      j = depth * i + q
      g_wait(q)
      if pre is not None:
        pre(j)
      pltpu.async_copy(bufs[q], acc.at[sidx.at[j]], ssems[q], add=True)
      q2 = (q + 2) % depth
      pl.when(j >= depth - 2)(lambda q2=q2: s_wait(q2))
      pl.when(j + 2 < nch)(lambda j=j, q2=q2: g_issue(j + 2, q2))
      if post is not None:
        post(j)
    return carry

  lax.fori_loop(0, nch // depth, body, 0)
  for dq in range(depth - 2, 0, -1):
    s_wait((nch - dq) % depth)


def _stage1_body(x_hbm, v4_hbm, e4_hbm, zrow_hbm, zflat_hbm,
                 outp_hbm, outce_hbm, outcv_hbm,
                 acc, vloc, eloc, b0, b1, b2, b3, b4, cnt_e, cnt_v,
                 g0, g1, g2, g3, g4, s0, s1, s2, s3, s4):
  cid = lax.axis_index("c")
  sid = lax.axis_index("s")
  wid = cid * NS + sid
  bufs = (b0, b1, b2, b3, b4)
  gsems = (g0, g1, g2, g3, g4)
  ssems = (s0, s1, s2, s3, s4)
  sl = pl.ds(sid * E_ROWS, E_ROWS)
  pltpu.sync_copy(zrow_hbm.at[pl.ds(0, E_ROWS)], acc.at[sl])
  pltpu.sync_copy(zflat_hbm.at[pl.ds(0, E_PAD)], cnt_e)
  pltpu.sync_copy(zflat_hbm, cnt_v)
  plsc.subcore_barrier()

  ones16 = jnp.ones((16,), jnp.float32)

  def hist(j):
    for t in range(K // 16):
      tsl = pl.ds(t * 16, 16)
      plsc.addupdate_scatter(cnt_e, [eloc[j, tsl]], ones16)
      plsc.addupdate_scatter(cnt_v, [vloc[j, tsl]], ones16)

  def sbody(sc, carry):
    pltpu.sync_copy(v4_hbm.at[wid, sc], vloc)
    pltpu.sync_copy(e4_hbm.at[wid, sc], eloc)
    _ring_loop(x_hbm, vloc, eloc, acc, bufs, gsems, ssems, SCH, post=hist)
    return carry

  lax.fori_loop(0, NSC1, sbody, 0)
  plsc.subcore_barrier()
  pltpu.sync_copy(acc.at[sl], outp_hbm.at[cid, sl])
  pltpu.sync_copy(cnt_e, outce_hbm.at[cid, sid])
  pltpu.sync_copy(cnt_v, outcv_hbm.at[cid, sid])


@functools.lru_cache(maxsize=None)
def _make_stage1():
  return functools.partial(
      pl.kernel,
      mesh=_get_mesh(),
      compiler_params=_SC_PARAMS,
      out_type=(
          jax.ShapeDtypeStruct((NC, E_PAD, D), jnp.float32),
          jax.ShapeDtypeStruct((NC, NS, E_PAD), jnp.float32),
          jax.ShapeDtypeStruct((NC, NS, N_PAD), jnp.float32),
      ),
      scratch_types=(
          [pltpu.VMEM_SHARED((E_PAD, D), jnp.float32),
           pltpu.VMEM((SCH, K), jnp.int32),
           pltpu.VMEM((SCH, K), jnp.int32)]
          + [pltpu.VMEM((K, D), jnp.float32)] * DEPTH
          + [pltpu.VMEM((E_PAD,), jnp.float32),
             pltpu.VMEM((N_PAD,), jnp.float32)]
          + [pltpu.SemaphoreType.DMA] * (2 * DEPTH)
      ),
  )(_stage1_body)


def _stage2_body(xe_hbm, e4_hbm, v4_hbm, zrow_hbm,
                 outp_hbm,
                 acc, gloc, sloc, lloc, b0, b1, b2, b3, b4,
                 g0, g1, g2, g3, g4, s0, s1, s2, s3, s4):
  cid = lax.axis_index("c")
  sid = lax.axis_index("s")
  lo = cid * HALF
  bufs = (b0, b1, b2, b3, b4)
  gsems = (g0, g1, g2, g3, g4)
  ssems = (s0, s1, s2, s3, s4)
  sl = pl.ds(sid * A_ROWS, A_ROWS)
  pltpu.sync_copy(zrow_hbm.at[pl.ds(0, A_ROWS)], acc.at[sl])
  plsc.subcore_barrier()

  def lcompute(j):
    for t in range(K // 16):
      tsl = pl.ds(t * 16, 16)
      v16 = sloc[j, tsl]
      inr = (v16 >= lo) & (v16 < lo + HALF)
      lloc[j, tsl] = jnp.where(inr, v16 - lo, HALF + (v16 & (TRASH - 1)))

  def sbody(sc, carry):
    pltpu.sync_copy(e4_hbm.at[sid, sc], gloc)
    pltpu.sync_copy(v4_hbm.at[sid, sc], sloc)
    _ring_loop(xe_hbm, gloc, lloc, acc, bufs, gsems, ssems, SCH, pre=lcompute)
    return carry

  lax.fori_loop(0, NSC2, sbody, 0)
  plsc.subcore_barrier()
  pltpu.sync_copy(acc.at[sl], outp_hbm.at[cid, sl])


@functools.lru_cache(maxsize=None)
def _make_stage2():
  return functools.partial(
      pl.kernel,
      mesh=_get_mesh(),
      compiler_params=_SC_PARAMS,
      out_type=jax.ShapeDtypeStruct((NC, AR, D), jnp.float32),
      scratch_types=(
          [pltpu.VMEM_SHARED((AR, D), jnp.float32),
           pltpu.VMEM((SCH, K), jnp.int32),
           pltpu.VMEM((SCH, K), jnp.int32),
           pltpu.VMEM((SCH, K), jnp.int32)]
          + [pltpu.VMEM((K, D), jnp.float32)] * DEPTH
          + [pltpu.SemaphoreType.DMA] * (2 * DEPTH)
      ),
  )(_stage2_body)


def _combine_body(p_ref, c_ref, o_ref):
  p = p_ref[0] + p_ref[1]
  c = jnp.sum(c_ref[...], axis=(0, 1))[:, None]
  o_ref[...] = p / jnp.maximum(c, 1.0)


def _combine(edge_p, edge_c):
  blk = 512
  grid = E_PAD // blk
  return pl.pallas_call(
      _combine_body,
      grid=(grid,),
      in_specs=[
          pl.BlockSpec((NC, blk, D), lambda i: (0, i, 0)),
          pl.BlockSpec((NC, NS, blk), lambda i: (0, 0, i)),
      ],
      out_specs=pl.BlockSpec((blk, D), lambda i: (i, 0)),
      out_shape=jax.ShapeDtypeStruct((E_PAD, D), jnp.float32),
  )(edge_p, edge_c)


def _linear_body(x_ref, w_ref, b_ref, y_ref):
  y_ref[...] = (jnp.dot(x_ref[...], w_ref[...],
                        preferred_element_type=jnp.float32) + b_ref[...])


def _linear(x, w_t, b2d):
  blk = 1000
  grid = N_NODES // blk
  return pl.pallas_call(
      _linear_body,
      grid=(grid,),
      in_specs=[
          pl.BlockSpec((blk, D), lambda i: (i, 0)),
          pl.BlockSpec((D, D), lambda i: (0, 0)),
          pl.BlockSpec((1, D), lambda i: (0, 0)),
      ],
      out_specs=pl.BlockSpec((blk, D), lambda i: (i, 0)),
      out_shape=jax.ShapeDtypeStruct((N_NODES, D), jnp.float32),
  )(x, w_t, b2d)


def _final_body(p_ref, c_ref, a_ref, x_ref, loss_ref):
  c = jnp.sum(c_ref[...], axis=(0, 1))[:, None]
  x = jnp.maximum(p_ref[0] / jnp.maximum(c, 1.0), 0.0)
  x_ref[...] = x

  blk_rows = x.shape[0]
  att = jnp.tile(a_ref[...], (blk_rows // HEADS, 1))
  a = jnp.sum(x * att, axis=1, keepdims=True) * (1.0 / D)
  a = jnp.where(a >= 0.0, a, 0.2 * a)
  s = jnp.clip(jax.nn.sigmoid(a), 0.01, 0.99)
  kl = s * jnp.log(s / 0.5) + (1.0 - s) * jnp.log((1.0 - s) / 0.5)
  part = jnp.sum(kl).reshape(1, 1)

  @pl.when(pl.program_id(0) == 0)
  def _():
    loss_ref[...] = jnp.zeros((1, 1), jnp.float32)

  loss_ref[...] += part


def _finalize(node_p, node_c, att):
  blk = 1024
  grid = N_PAD // blk
  return pl.pallas_call(
      _final_body,
      grid=(grid,),
      in_specs=[
          pl.BlockSpec((1, blk, D), lambda i: (i // (HALF // blk),
                                               i % (HALF // blk), 0)),
          pl.BlockSpec((NC, NS, blk), lambda i: (0, 0, i)),
          pl.BlockSpec((HEADS, D), lambda i: (0, 0)),
      ],
      out_specs=[
          pl.BlockSpec((blk, D), lambda i: (i, 0)),
          pl.BlockSpec((1, 1), lambda i: (0, 0)),
      ],
      out_shape=[
          jax.ShapeDtypeStruct((N_PAD, D), jnp.float32),
          jax.ShapeDtypeStruct((1, 1), jnp.float32),
      ],
  )(node_p, node_c, att)


def kernel(X, v_idx, e_idx, theta_W, theta_b, att):
  v_idx = v_idx.astype(jnp.int32)
  e_idx = e_idx.astype(jnp.int32)
  zrow = jnp.zeros((A_ROWS, D), jnp.float32)
  zflat = jnp.zeros((N_PAD,), jnp.float32)
  v31 = v_idx.reshape(NW, NSC1, SCH, K)
  e31 = e_idx.reshape(NW, NSC1, SCH, K)
  v32 = v_idx.reshape(NS, NSC2, SCH, K)
  e32 = e_idx.reshape(NS, NSC2, SCH, K)

  y = _linear(X, theta_W.T, theta_b[None, :])
  edge_p, edge_c, node_c = _make_stage1()(y, v31, e31, zrow, zflat)
  x_e = _combine(edge_p, edge_c)
  node_p = _make_stage2()(x_e, e32, v32, zrow)
  x_out, loss = _finalize(node_p, node_c, att)
  return x_out[:N_NODES], loss[0, 0]
